# baseline (speedup 1.0000x reference)
import functools

import jax
import jax.numpy as jnp
from jax import lax
from jax.experimental import pallas as pl
from jax.experimental.pallas import tpu as pltpu
from jax.experimental.pallas import tpu_sc as plsc

EMB = 300
EMB_PAD = 384
CHUNK = 128
NBUF = 2
PAD_BLOCK = 2048


def _pad_table(table):
    v = table.shape[0]
    grid = (v + PAD_BLOCK - 1) // PAD_BLOCK

    def body(t_ref, o_ref):
        o_ref[...] = jnp.concatenate(
            [t_ref[...], jnp.zeros((PAD_BLOCK, EMB_PAD - EMB), jnp.float32)],
            axis=1,
        )

    return pl.pallas_call(
        body,
        grid=(grid,),
        in_specs=[pl.BlockSpec((PAD_BLOCK, EMB), lambda i: (i, 0))],
        out_specs=pl.BlockSpec((PAD_BLOCK, EMB_PAD), lambda i: (i, 0)),
        out_shape=jax.ShapeDtypeStruct((v, EMB_PAD), jnp.float32),
    )(table)


SLICE_BLOCK = 4096


def _slice_out(out_pad):
    n_rows = out_pad.shape[0]
    assert n_rows % SLICE_BLOCK == 0

    def body(i_ref, o_ref):
        o_ref[...] = i_ref[:, :EMB]

    return pl.pallas_call(
        body,
        grid=(n_rows // SLICE_BLOCK,),
        in_specs=[pl.BlockSpec((SLICE_BLOCK, EMB_PAD), lambda i: (i, 0))],
        out_specs=pl.BlockSpec((SLICE_BLOCK, EMB), lambda i: (i, 0)),
        out_shape=jax.ShapeDtypeStruct((n_rows, EMB), jnp.float32),
    )(out_pad)


def _emb_kernel(n_rows):
    info = plsc.get_sparse_core_info()
    nc, ns = info.num_cores, info.num_subcores
    nw = nc * ns
    assert n_rows % (nw * CHUNK) == 0
    t_per_w = n_rows // (nw * CHUNK)
    assert t_per_w % NBUF == 0

    mesh = plsc.VectorSubcoreMesh(core_axis_name="c", subcore_axis_name="s")

    @functools.partial(
        pl.kernel,
        mesh=mesh,
        compiler_params=pltpu.CompilerParams(use_tc_tiling_on_sc=True),
        out_type=jax.ShapeDtypeStruct((n_rows, EMB_PAD), jnp.float32),
        scratch_types=[
            pltpu.VMEM((NBUF, CHUNK), jnp.int32),
            pltpu.VMEM((NBUF, CHUNK, EMB_PAD), jnp.float32),
            pltpu.SemaphoreType.DMA,
            pltpu.SemaphoreType.DMA,
        ],
    )
    def k(tok_hbm, table_hbm, out_hbm, idx_v, rows_v, sem0, sem1):
        sems = (sem0, sem1)
        wid = lax.axis_index("s") * nc + lax.axis_index("c")
        base = wid * t_per_w

        def prime(chunk_id, b):
            pltpu.sync_copy(tok_hbm.at[chunk_id], idx_v.at[b])
            pltpu.async_copy(table_hbm.at[idx_v.at[b]], rows_v.at[b], sems[b])

        def drain(chunk_id, b):
            pltpu.make_async_copy(
                table_hbm.at[idx_v.at[b]], rows_v.at[b], sems[b]
            ).wait()
            pltpu.sync_copy(
                rows_v.at[b], out_hbm.at[pl.ds(chunk_id * CHUNK, CHUNK)]
            )

        for b in range(NBUF):
            prime(base + b, b)

        def body(j, carry):
            for b in range(NBUF):
                i = base + NBUF * j + b
                drain(i, b)
                prime(i + NBUF, b)
            return carry

        lax.fori_loop(0, t_per_w // NBUF - 1, body, 0, unroll=False)

        for b in range(NBUF):
            drain(base + t_per_w - NBUF + b, b)

    return k


def kernel(tokens, table):
    b, l = tokens.shape
    n_rows = b * l
    tok_flat = tokens.astype(jnp.int32).reshape(n_rows // CHUNK, CHUNK)
    table_pad = _pad_table(table)
    out = _emb_kernel(n_rows)(tok_flat, table_pad)
    return _slice_out(out).reshape(b, l, EMB)


# speedup vs baseline: 1.4029x; 1.4029x over previous
import functools

import jax
import jax.numpy as jnp
from jax import lax
from jax.experimental import pallas as pl
from jax.experimental.pallas import tpu as pltpu
from jax.experimental.pallas import tpu_sc as plsc

EMB = 300
EMB_PAD = 384
CHUNK = 128
NBUF = 2
PAD_BLOCK = 2048


def _pad_table(table):
    v = table.shape[0]
    grid = (v + PAD_BLOCK - 1) // PAD_BLOCK

    def body(t_ref, o_ref):
        o_ref[...] = jnp.concatenate(
            [t_ref[...], jnp.zeros((PAD_BLOCK, EMB_PAD - EMB), jnp.float32)],
            axis=1,
        )

    return pl.pallas_call(
        body,
        grid=(grid,),
        in_specs=[pl.BlockSpec((PAD_BLOCK, EMB), lambda i: (i, 0))],
        out_specs=pl.BlockSpec((PAD_BLOCK, EMB_PAD), lambda i: (i, 0)),
        out_shape=jax.ShapeDtypeStruct((v, EMB_PAD), jnp.float32),
    )(table)


def _emb_kernel(n_rows):
    info = plsc.get_sparse_core_info()
    nc, ns = info.num_cores, info.num_subcores
    nw = nc * ns
    assert n_rows % (nw * CHUNK) == 0
    t_per_w = n_rows // (nw * CHUNK)
    assert t_per_w % NBUF == 0

    mesh = plsc.VectorSubcoreMesh(core_axis_name="c", subcore_axis_name="s")

    @functools.partial(
        pl.kernel,
        mesh=mesh,
        compiler_params=pltpu.CompilerParams(use_tc_tiling_on_sc=True),
        out_type=jax.ShapeDtypeStruct((n_rows, EMB_PAD), jnp.float32),
        scratch_types=[
            pltpu.VMEM((NBUF, CHUNK), jnp.int32),
            pltpu.VMEM((NBUF, CHUNK, EMB_PAD), jnp.float32),
            pltpu.SemaphoreType.DMA,
            pltpu.SemaphoreType.DMA,
        ],
    )
    def k(tok_hbm, table_hbm, out_hbm, idx_v, rows_v, sem0, sem1):
        sems = (sem0, sem1)
        wid = lax.axis_index("s") * nc + lax.axis_index("c")
        base = wid * t_per_w

        def prime(chunk_id, b):
            pltpu.sync_copy(tok_hbm.at[chunk_id], idx_v.at[b])
            pltpu.async_copy(table_hbm.at[idx_v.at[b]], rows_v.at[b], sems[b])

        def drain(chunk_id, b):
            pltpu.make_async_copy(
                table_hbm.at[idx_v.at[b]], rows_v.at[b], sems[b]
            ).wait()
            pltpu.sync_copy(
                rows_v.at[b], out_hbm.at[pl.ds(chunk_id * CHUNK, CHUNK)]
            )

        for b in range(NBUF):
            prime(base + b, b)

        def body(j, carry):
            for b in range(NBUF):
                i = base + NBUF * j + b
                drain(i, b)
                prime(i + NBUF, b)
            return carry

        lax.fori_loop(0, t_per_w // NBUF - 1, body, 0, unroll=False)

        for b in range(NBUF):
            drain(base + t_per_w - NBUF + b, b)

    return k


def kernel(tokens, table):
    b, l = tokens.shape
    n_rows = b * l
    tok_flat = tokens.astype(jnp.int32).reshape(n_rows // CHUNK, CHUNK)
    table_pad = _pad_table(table)
    out = _emb_kernel(n_rows)(tok_flat, table_pad)
    return out[:, :EMB].reshape(b, l, EMB)
